# baseline (speedup 1.0000x reference)
import functools

import jax
import jax.numpy as jnp
from jax import lax
from jax.experimental import pallas as pl
from jax.experimental.pallas import tpu as pltpu
from jax.experimental.pallas import tpu_sc as plsc

N = 10000
D = 128
E = 320000

NW = 32
C = 128
NCH = 79
EPW = NCH * C
EPAD = NW * EPW
NACC = NCH * C
NDNL = N + 16

_DOT = functools.partial(
    lax.dot_general,
    dimension_numbers=(((1,), (0,)), ((), ())),
    precision=lax.Precision.HIGHEST,
    preferred_element_type=jnp.float32,
)


def _dense_norm_body(q_ref, dn_ref, w_ref, b_ref, as_ref, ad_ref, flag_ref,
                     h_ref, s_ref, d_ref):
    qsum = q_ref[0] + q_ref[1]
    dn = jnp.sum(dn_ref[0], axis=0)
    dnc = dn[:, None]
    xin = jnp.where(dnc > 0, qsum / dnc, 0.0)
    xin = jnp.where(flag_ref[0, 0] > 0, jnp.maximum(xin, 0.0), xin)
    h = _DOT(xin, w_ref[...]) + b_ref[...]
    h_ref[...] = h
    s_ref[...] = _DOT(h, as_ref[...])
    d_ref[...] = _DOT(h, ad_ref[...])


def _final_body(q_ref, dn_ref, out_ref):
    qsum = q_ref[0] + q_ref[1]
    dn = jnp.sum(dn_ref[0], axis=0)
    dnc = dn[:, None]
    out_ref[...] = jnp.where(dnc > 0, qsum / dnc, 0.0)


_ROWS = 1000


def _dense_norm_layer(q, dn, W, b, asv, adv, flag):
    return pl.pallas_call(
        _dense_norm_body,
        grid=(N // _ROWS,),
        in_specs=[
            pl.BlockSpec((2, _ROWS, D), lambda i: (0, i, 0)),
            pl.BlockSpec((1, NW, _ROWS), lambda i: (i, 0, 0)),
            pl.BlockSpec((D, D), lambda i: (0, 0)),
            pl.BlockSpec((1, D), lambda i: (0, 0)),
            pl.BlockSpec((D, 1), lambda i: (0, 0)),
            pl.BlockSpec((D, 1), lambda i: (0, 0)),
            pl.BlockSpec((1, 1), lambda i: (0, 0)),
        ],
        out_specs=[
            pl.BlockSpec((_ROWS, D), lambda i: (i, 0)),
            pl.BlockSpec((_ROWS, 1), lambda i: (i, 0)),
            pl.BlockSpec((_ROWS, 1), lambda i: (i, 0)),
        ],
        out_shape=[
            jax.ShapeDtypeStruct((N, D), jnp.float32),
            jax.ShapeDtypeStruct((N, 1), jnp.float32),
            jax.ShapeDtypeStruct((N, 1), jnp.float32),
        ],
    )(q, dn, W, b[None, :], asv[:, None], adv[:, None],
      flag.reshape(1, 1))


def _finalize(q, dn):
    return pl.pallas_call(
        _final_body,
        grid=(N // _ROWS,),
        in_specs=[
            pl.BlockSpec((2, _ROWS, D), lambda i: (0, i, 0)),
            pl.BlockSpec((1, NW, _ROWS), lambda i: (i, 0, 0)),
        ],
        out_specs=pl.BlockSpec((_ROWS, D), lambda i: (i, 0)),
        out_shape=jax.ShapeDtypeStruct((N, D), jnp.float32),
    )(q, dn)


RS = 5
RW = 3
RR = 2


def _edge_body(src_hbm, dst_hbm, w_hbm, dst3_hbm, s_hbm, d_hbm, h_hbm,
               q_hbm, dn_hbm,
               dnl_v, rows_v, src_r, dst_r, w_r, sv_r, dv_r, dstw_r,
               acc_sh, stsem, g2sem, gsem, csem):
    c = lax.axis_index("c")
    sub = lax.axis_index("s")
    wid = c * 16 + sub

    zv = jnp.zeros((16,), jnp.float32)

    def _zero_dn(i, _):
        dnl_v[pl.ds(pl.multiple_of(i * 16, 8), 16)] = zv
        return 0

    lax.fori_loop(0, NDNL // 16, _zero_dn, 0)

    def _zero_rows(i, _):
        for t in range(8):
            rows_v[0, i, pl.ds(t * 16, 16)] = zv
        return 0

    lax.fori_loop(0, C, _zero_rows, 0)
    for i in range(5):
        bid = sub * 5 + i

        @pl.when(bid < NACC // C)
        def _():
            off = pl.multiple_of(bid * C, 8)
            pltpu.sync_copy(rows_v.at[0], acc_sh.at[pl.ds(off, C)])

    plsc.subcore_barrier()

    base = wid * EPW

    def _rsl(slot):
        return pl.ds(pl.multiple_of(slot * C, 8), C)

    def _stage(ch, slot):
        off = pl.multiple_of(base + ch * C, 8)
        sl = _rsl(slot)
        pltpu.async_copy(src_hbm.at[pl.ds(off, C)], src_r.at[sl], stsem)
        pltpu.async_copy(dst_hbm.at[pl.ds(off, C)], dst_r.at[sl], stsem)
        pltpu.async_copy(w_hbm.at[pl.ds(off, C)], w_r.at[sl], stsem)

    def _stage_wait(ch, slot):
        off = pl.multiple_of(base + ch * C, 8)
        sl = _rsl(slot)
        pltpu.make_async_copy(src_hbm.at[pl.ds(off, C)], src_r.at[sl],
                              stsem).wait()
        pltpu.make_async_copy(dst_hbm.at[pl.ds(off, C)], dst_r.at[sl],
                              stsem).wait()
        pltpu.make_async_copy(w_hbm.at[pl.ds(off, C)], w_r.at[sl],
                              stsem).wait()

    def _stage_w(ch, slotw):
        cid = wid * NCH + ch
        pltpu.async_copy(dst3_hbm.at[cid], dstw_r.at[slotw], stsem)

    def _stage_w_wait(ch, slotw):
        cid = wid * NCH + ch
        pltpu.make_async_copy(dst3_hbm.at[cid], dstw_r.at[slotw],
                              stsem).wait()

    def _sgather(slot):
        sl = _rsl(slot)
        pltpu.async_copy(s_hbm.at[src_r.at[sl]], sv_r.at[sl], g2sem)
        pltpu.async_copy(d_hbm.at[dst_r.at[sl]], dv_r.at[sl], g2sem)

    def _sgather_wait(slot):
        sl = _rsl(slot)
        pltpu.make_async_copy(s_hbm.at[src_r.at[sl]], sv_r.at[sl],
                              g2sem).wait()
        pltpu.make_async_copy(d_hbm.at[dst_r.at[sl]], dv_r.at[sl],
                              g2sem).wait()

    def _rgather(ch, q):
        pltpu.async_copy(h_hbm.at[src_r.at[_rsl(lax.rem(ch, RS))]],
                         rows_v.at[q], gsem)

    def _rgather_wait(ch, q):
        pltpu.make_async_copy(h_hbm.at[src_r.at[_rsl(lax.rem(ch, RS))]],
                              rows_v.at[q], gsem).wait()

    def _scat_drain(ch):
        q = lax.rem(ch, RR)
        slotw = lax.rem(ch, RW)
        pltpu.make_async_copy(rows_v.at[q],
                              acc_sh.at[dstw_r.at[slotw].at[0]],
                              csem).wait()

    for ch0 in range(RS):
        _stage(ch0, ch0)
    for ch0 in range(3):
        _stage_wait(ch0, ch0)
        _sgather(ch0)
    _stage_w(0, 0)
    _stage_w(1, 1)
    _rgather(0, 0)

    def _chunk(ch, _):
        slot = lax.rem(ch, RS)
        q = lax.rem(ch, RR)
        slotw = lax.rem(ch, RW)

        @pl.when(ch >= 1)
        def _():
            _scat_drain(ch - 1)

        @pl.when(ch + 2 < NCH)
        def _():
            _stage_w(ch + 2, lax.rem(ch + 2, RW))

        @pl.when(ch + 1 < NCH)
        def _():
            _rgather(ch + 1, lax.rem(ch + 1, RR))

        _sgather_wait(slot)
        rbase = pl.multiple_of(slot * C, 8)
        evs = []
        for g in range(C // 16):
            sl = pl.ds(rbase + g * 16, 16)
            di = dst_r[sl]
            a = sv_r[sl] + dv_r[sl]
            a = jnp.where(a >= 0, a, 0.2 * a) * w_r[sl]
            ev = jnp.exp(a)
            evs.append(ev)
            plsc.addupdate_scatter(dnl_v, [di], ev)

        _rgather_wait(ch, q)
        for g in range(C // 16):
            for k in range(16):
                coef = evs[g][k]
                j = g * 16 + k
                for t in range(8):
                    tsl = pl.ds(t * 16, 16)
                    rows_v[q, j, tsl] = rows_v[q, j, tsl] * coef
        _stage_w_wait(ch, slotw)
        pltpu.async_copy(rows_v.at[q], acc_sh.at[dstw_r.at[slotw].at[0]],
                         csem, add=True)

        @pl.when(ch + RS < NCH)
        def _():
            _stage(ch + RS, slot)

        @pl.when(ch + 3 < NCH)
        def _():
            nslot = lax.rem(ch + 3, RS)
            _stage_wait(ch + 3, nslot)
            _sgather(nslot)

        return 0

    lax.fori_loop(0, NCH, _chunk, 0)
    _scat_drain(NCH - 1)
    plsc.subcore_barrier()

    for i in range(5):
        bid = sub * 5 + i

        @pl.when(bid < NACC // C)
        def _():
            off = pl.multiple_of(bid * C, 8)
            pltpu.sync_copy(acc_sh.at[pl.ds(off, C)],
                            q_hbm.at[c].at[pl.ds(off, C)])

    for blk in range(N // 1000):
        dn_off = pl.multiple_of(blk * NW * 1000 + wid * 1000, 8)
        pltpu.sync_copy(dnl_v.at[pl.ds(blk * 1000, 1000)],
                        dn_hbm.at[pl.ds(dn_off, 1000)])


_edge_pass = pl.kernel(
    _edge_body,
    out_type=[
        jax.ShapeDtypeStruct((2, NACC, D), jnp.float32),
        jax.ShapeDtypeStruct((NW * N,), jnp.float32),
    ],
    mesh=plsc.VectorSubcoreMesh(core_axis_name="c", subcore_axis_name="s"),
    compiler_params=pltpu.CompilerParams(needs_layout_passes=False),
    scratch_types=[
        pltpu.VMEM((NDNL,), jnp.float32),
        pltpu.VMEM((RR, C, D), jnp.float32),
        pltpu.VMEM((RS * C,), jnp.int32),
        pltpu.VMEM((RS * C,), jnp.int32),
        pltpu.VMEM((RS * C,), jnp.float32),
        pltpu.VMEM((RS * C,), jnp.float32),
        pltpu.VMEM((RS * C,), jnp.float32),
        pltpu.VMEM((RW, 1, C), jnp.int32),
        pltpu.VMEM_SHARED((NACC, D), jnp.float32),
        pltpu.SemaphoreType.DMA,
        pltpu.SemaphoreType.DMA,
        pltpu.SemaphoreType.DMA,
        pltpu.SemaphoreType.DMA,
    ],
)


def kernel(x, edge_index, edge_weight, W1, b1, as1, ad1, W2, b2, as2, ad2):
    npad = EPAD - E
    srcf = jnp.concatenate(
        [edge_index[0].astype(jnp.int32), jnp.zeros(npad, jnp.int32)])
    dstf = jnp.concatenate(
        [edge_index[1].astype(jnp.int32), jnp.full(npad, N + 8, jnp.int32)])
    wf = jnp.concatenate([edge_weight, jnp.zeros(npad, jnp.float32)])
    dst3 = jnp.concatenate(
        [edge_index[1].astype(jnp.int32), jnp.full(npad, N, jnp.int32)]
    ).reshape(EPAD // C, 1, C)

    xpad = jnp.concatenate([x, jnp.zeros((NACC - N, D), jnp.float32)])
    q0 = jnp.stack([xpad, jnp.zeros_like(xpad)])
    dn0 = jnp.full((NW * N,), 1.0 / NW, jnp.float32)
    Ws = jnp.stack([W1, W2])
    bs = jnp.stack([b1, b2])
    avs = jnp.stack([as1, as2])
    avd = jnp.stack([ad1, ad2])
    flags = jnp.array([0.0, 1.0], jnp.float32)

    def _layer(carry, xs):
        q, dn = carry
        W, b, asv, adv, flag = xs
        h, s, d = _dense_norm_layer(q, dn.reshape(N // _ROWS, NW, _ROWS),
                                    W, b, asv, adv, flag)
        qn, dnn = _edge_pass(srcf, dstf, wf, dst3, s[:, 0],
                             jnp.concatenate([d[:, 0], jnp.zeros(16, jnp.float32)]),
                             h)
        return (qn, dnn), None

    (qf, dnf), _ = lax.scan(_layer, (q0, dn0), (Ws, bs, avs, avd, flags))
    return _finalize(qf, dnf.reshape(N // _ROWS, NW, _ROWS))


# speedup vs baseline: 1.7801x; 1.7801x over previous
import functools

import jax
import jax.numpy as jnp
from jax import lax
from jax.experimental import pallas as pl
from jax.experimental.pallas import tpu as pltpu
from jax.experimental.pallas import tpu_sc as plsc

N = 10000
D = 128
E = 320000

NW = 32
EPW = E // NW
C = 80
NCH = EPW // C

_DOT = functools.partial(
    lax.dot_general,
    dimension_numbers=(((1,), (0,)), ((), ())),
    precision=lax.Precision.HIGHEST,
    preferred_element_type=jnp.float32,
)


def _dense_norm_body(q_ref, dn_ref, w_ref, b_ref, as_ref, ad_ref, flag_ref,
                     h_ref, s_ref, d_ref):
    qsum = q_ref[0] + q_ref[1]
    dn = jnp.sum(dn_ref[0], axis=0)
    dnc = dn[:, None]
    xin = jnp.where(dnc > 0, qsum / dnc, 0.0)
    xin = jnp.where(flag_ref[0, 0] > 0, jnp.maximum(xin, 0.0), xin)
    h = _DOT(xin, w_ref[...]) + b_ref[...]
    h_ref[...] = h
    s_ref[...] = _DOT(h, as_ref[...])
    d_ref[...] = _DOT(h, ad_ref[...])


def _final_body(q_ref, dn_ref, out_ref):
    qsum = q_ref[0] + q_ref[1]
    dn = jnp.sum(dn_ref[0], axis=0)
    dnc = dn[:, None]
    out_ref[...] = jnp.where(dnc > 0, qsum / dnc, 0.0)


_ROWS = 1000


def _dense_norm_layer(q, dn, W, b, asv, adv, flag):
    return pl.pallas_call(
        _dense_norm_body,
        grid=(N // _ROWS,),
        in_specs=[
            pl.BlockSpec((2, _ROWS, D), lambda i: (0, i, 0)),
            pl.BlockSpec((1, NW, _ROWS), lambda i: (i, 0, 0)),
            pl.BlockSpec((D, D), lambda i: (0, 0)),
            pl.BlockSpec((1, D), lambda i: (0, 0)),
            pl.BlockSpec((D, 1), lambda i: (0, 0)),
            pl.BlockSpec((D, 1), lambda i: (0, 0)),
            pl.BlockSpec((1, 1), lambda i: (0, 0)),
        ],
        out_specs=[
            pl.BlockSpec((_ROWS, D), lambda i: (i, 0)),
            pl.BlockSpec((_ROWS, 1), lambda i: (i, 0)),
            pl.BlockSpec((_ROWS, 1), lambda i: (i, 0)),
        ],
        out_shape=[
            jax.ShapeDtypeStruct((N, D), jnp.float32),
            jax.ShapeDtypeStruct((N, 1), jnp.float32),
            jax.ShapeDtypeStruct((N, 1), jnp.float32),
        ],
    )(q, dn, W, b[None, :], asv[:, None], adv[:, None],
      flag.reshape(1, 1))


def _finalize(q, dn):
    return pl.pallas_call(
        _final_body,
        grid=(N // _ROWS,),
        in_specs=[
            pl.BlockSpec((2, _ROWS, D), lambda i: (0, i, 0)),
            pl.BlockSpec((1, NW, _ROWS), lambda i: (i, 0, 0)),
        ],
        out_specs=pl.BlockSpec((_ROWS, D), lambda i: (i, 0)),
        out_shape=jax.ShapeDtypeStruct((N, D), jnp.float32),
    )(q, dn)


RS = 6
RW = 4
RR = 3


def _edge_body(src_hbm, dst_hbm, w_hbm, dst3_hbm, s_hbm, d_hbm, h_hbm,
               q_hbm, dn_hbm,
               dnl_v, rows_v, src_r, dst_r, w_r, sv_r, dv_r, dstw_r,
               acc_sh, stsem, g2sem, gsem, csem):
    c = lax.axis_index("c")
    sub = lax.axis_index("s")
    wid = c * 16 + sub

    zv = jnp.zeros((16,), jnp.float32)

    def _zero_dn(i, _):
        dnl_v[pl.ds(pl.multiple_of(i * 16, 8), 16)] = zv
        return 0

    lax.fori_loop(0, N // 16, _zero_dn, 0)

    def _zero_rows(i, _):
        for t in range(8):
            rows_v[0, i, pl.ds(t * 16, 16)] = zv
        return 0

    lax.fori_loop(0, C, _zero_rows, 0)
    for i in range(8):
        bid = sub * 8 + i

        @pl.when(bid < N // C)
        def _():
            off = pl.multiple_of(bid * C, 8)
            pltpu.sync_copy(rows_v.at[0], acc_sh.at[pl.ds(off, C)])

    plsc.subcore_barrier()

    base = wid * EPW

    def _rsl(slot):
        return pl.ds(pl.multiple_of(slot * C, 8), C)

    def _stage(ch, slot):
        off = pl.multiple_of(base + ch * C, 8)
        sl = _rsl(slot)
        pltpu.async_copy(src_hbm.at[pl.ds(off, C)], src_r.at[sl], stsem)
        pltpu.async_copy(dst_hbm.at[pl.ds(off, C)], dst_r.at[sl], stsem)
        pltpu.async_copy(w_hbm.at[pl.ds(off, C)], w_r.at[sl], stsem)

    def _stage_wait(ch, slot):
        off = pl.multiple_of(base + ch * C, 8)
        sl = _rsl(slot)
        pltpu.make_async_copy(src_hbm.at[pl.ds(off, C)], src_r.at[sl],
                              stsem).wait()
        pltpu.make_async_copy(dst_hbm.at[pl.ds(off, C)], dst_r.at[sl],
                              stsem).wait()
        pltpu.make_async_copy(w_hbm.at[pl.ds(off, C)], w_r.at[sl],
                              stsem).wait()

    def _stage_w(ch, slotw):
        cid = wid * NCH + ch
        pltpu.async_copy(dst3_hbm.at[cid], dstw_r.at[slotw], stsem)

    def _stage_w_wait(ch, slotw):
        cid = wid * NCH + ch
        pltpu.make_async_copy(dst3_hbm.at[cid], dstw_r.at[slotw],
                              stsem).wait()

    def _sgather(slot):
        sl = _rsl(slot)
        pltpu.async_copy(s_hbm.at[src_r.at[sl]], sv_r.at[sl], g2sem)
        pltpu.async_copy(d_hbm.at[dst_r.at[sl]], dv_r.at[sl], g2sem)

    def _sgather_wait(slot):
        sl = _rsl(slot)
        pltpu.make_async_copy(s_hbm.at[src_r.at[sl]], sv_r.at[sl],
                              g2sem).wait()
        pltpu.make_async_copy(d_hbm.at[dst_r.at[sl]], dv_r.at[sl],
                              g2sem).wait()

    def _rgather(ch, q):
        pltpu.async_copy(h_hbm.at[src_r.at[_rsl(lax.rem(ch, RS))]],
                         rows_v.at[q], gsem)

    def _rgather_wait(ch, q):
        pltpu.make_async_copy(h_hbm.at[src_r.at[_rsl(lax.rem(ch, RS))]],
                              rows_v.at[q], gsem).wait()

    def _scat_drain(ch):
        q = lax.rem(ch, RR)
        slotw = lax.rem(ch, RW)
        pltpu.make_async_copy(rows_v.at[q],
                              acc_sh.at[dstw_r.at[slotw].at[0]],
                              csem).wait()

    for ch0 in range(RS):
        _stage(ch0, ch0)
    for ch0 in range(4):
        _stage_wait(ch0, ch0)
        _sgather(ch0)
    _stage_w(0, 0)
    _stage_w(1, 1)
    _rgather(0, 0)

    def _chunk(ch, _):
        slot = lax.rem(ch, RS)
        q = lax.rem(ch, RR)
        slotw = lax.rem(ch, RW)

        @pl.when(ch >= 2)
        def _():
            _scat_drain(ch - 2)

        @pl.when(ch + 2 < NCH)
        def _():
            _stage_w(ch + 2, lax.rem(ch + 2, RW))

        @pl.when(ch + 1 < NCH)
        def _():
            _rgather(ch + 1, lax.rem(ch + 1, RR))

        _sgather_wait(slot)
        rbase = pl.multiple_of(slot * C, 8)
        evs = []
        for g in range(C // 16):
            sl = pl.ds(rbase + g * 16, 16)
            di = dst_r[sl]
            a = sv_r[sl] + dv_r[sl]
            a = jnp.where(a >= 0, a, 0.2 * a) * w_r[sl]
            ev = jnp.exp(a)
            evs.append(ev)
            plsc.addupdate_scatter(dnl_v, [di], ev)

        _rgather_wait(ch, q)
        for g in range(C // 16):
            for k in range(16):
                coef = evs[g][k]
                j = g * 16 + k
                for t in range(8):
                    tsl = pl.ds(t * 16, 16)
                    rows_v[q, j, tsl] = rows_v[q, j, tsl] * coef
        _stage_w_wait(ch, slotw)
        pltpu.async_copy(rows_v.at[q], acc_sh.at[dstw_r.at[slotw].at[0]],
                         csem, add=True)

        @pl.when(ch + RS < NCH)
        def _():
            _stage(ch + RS, slot)

        @pl.when(ch + 4 < NCH)
        def _():
            nslot = lax.rem(ch + 4, RS)
            _stage_wait(ch + 4, nslot)
            _sgather(nslot)

        return 0

    lax.fori_loop(0, NCH, _chunk, 0)
    _scat_drain(NCH - 2)
    _scat_drain(NCH - 1)
    plsc.subcore_barrier()

    for i in range(8):
        bid = sub * 8 + i

        @pl.when(bid < N // C)
        def _():
            off = pl.multiple_of(bid * C, 8)
            pltpu.sync_copy(acc_sh.at[pl.ds(off, C)],
                            q_hbm.at[c].at[pl.ds(off, C)])

    for blk in range(N // 1000):
        dn_off = pl.multiple_of(blk * NW * 1000 + wid * 1000, 8)
        pltpu.sync_copy(dnl_v.at[pl.ds(blk * 1000, 1000)],
                        dn_hbm.at[pl.ds(dn_off, 1000)])


_edge_pass = pl.kernel(
    _edge_body,
    out_type=[
        jax.ShapeDtypeStruct((2, N, D), jnp.float32),
        jax.ShapeDtypeStruct((NW * N,), jnp.float32),
    ],
    mesh=plsc.VectorSubcoreMesh(core_axis_name="c", subcore_axis_name="s"),
    compiler_params=pltpu.CompilerParams(needs_layout_passes=False),
    scratch_types=[
        pltpu.VMEM((N,), jnp.float32),
        pltpu.VMEM((RR, C, D), jnp.float32),
        pltpu.VMEM((RS * C,), jnp.int32),
        pltpu.VMEM((RS * C,), jnp.int32),
        pltpu.VMEM((RS * C,), jnp.float32),
        pltpu.VMEM((RS * C,), jnp.float32),
        pltpu.VMEM((RS * C,), jnp.float32),
        pltpu.VMEM((RW, 1, C), jnp.int32),
        pltpu.VMEM_SHARED((N, D), jnp.float32),
        pltpu.SemaphoreType.DMA,
        pltpu.SemaphoreType.DMA,
        pltpu.SemaphoreType.DMA,
        pltpu.SemaphoreType.DMA,
    ],
)


def kernel(x, edge_index, edge_weight, W1, b1, as1, ad1, W2, b2, as2, ad2):
    srcf = edge_index[0].astype(jnp.int32)
    dstf = edge_index[1].astype(jnp.int32)
    dst3 = dstf.reshape(E // C, 1, C)

    q0 = jnp.stack([x, jnp.zeros_like(x)])
    dn0 = jnp.full((NW * N,), 1.0 / NW, jnp.float32)
    Ws = jnp.stack([W1, W2])
    bs = jnp.stack([b1, b2])
    avs = jnp.stack([as1, as2])
    avd = jnp.stack([ad1, ad2])
    flags = jnp.array([0.0, 1.0], jnp.float32)

    def _layer(carry, xs):
        q, dn = carry
        W, b, asv, adv, flag = xs
        h, s, d = _dense_norm_layer(q, dn.reshape(N // _ROWS, NW, _ROWS),
                                    W, b, asv, adv, flag)
        qn, dnn = _edge_pass(srcf, dstf, edge_weight, dst3,
                             s[:, 0], d[:, 0], h)
        return (qn, dnn), None

    (qf, dnf), _ = lax.scan(_layer, (q0, dn0), (Ws, bs, avs, avd, flags))
    return _finalize(qf, dnf.reshape(N // _ROWS, NW, _ROWS))
